# Initial kernel scaffold; baseline (speedup 1.0000x reference)
import jax
import jax.numpy as jnp
from jax.experimental import pallas as pl


def kernel(obs, actions, rnn_hxs, embed1, Wih, Whh, bih, bhh, We2, be2, Wsh, bsh, Wcr, bcr, Wm1, bm1, Wm2, bm2):
    raise NotImplementedError("write your pallas kernel here")


# speedup vs baseline: 5.5585x; 5.5585x over previous
import jax
import jax.numpy as jnp
from jax.experimental import pallas as pl
from jax.experimental.pallas import tpu as pltpu

T, N = 16, 1024
E, Hd, A = 64, 128, 32
Ps, Ph = 16, 16
V = 64
INF = 1e8
TN = 256


def _select_action_block(Y, P):
    blk = jax.lax.broadcasted_iota(jnp.int32, (TN, A * Hd), 1) // Hd
    Yf = jnp.where(blk == P, Y, 0.0)
    w = A * Hd
    while w > Hd:
        w //= 2
        Yf = Yf[:, :w] + Yf[:, w:]
    return Yf


def _body(obs_ref, embed1_ref, WihT_ref, bih_ref, bhh_ref, We2T_ref, be2_ref,
          Wsh_ref, bsh_ref, WcrT_ref, bcr_ref, W1all_ref, bm1_ref, Wm2T_ref,
          bm2_ref, sl_out, X_out, vals_out, ml_out, el_out):
    f32 = jnp.float32
    obs = obs_ref[...]
    embed1 = embed1_ref[...]
    iotaV = jax.lax.broadcasted_iota(jnp.int32, (TN, V), 1)
    emb = []
    for t in range(T):
        oh_t = (obs[:, t:t + 1] == iotaV).astype(f32)
        emb.append(jnp.dot(oh_t, embed1, preferred_element_type=f32))

    bhh = bhh_ref[...]
    gi = jnp.dot(emb[0], WihT_ref[...], preferred_element_type=f32) + bih_ref[...]
    r = jax.nn.sigmoid(gi[:, :Hd] + bhh[:, :Hd])
    z = jax.nn.sigmoid(gi[:, Hd:2 * Hd] + bhh[:, Hd:2 * Hd])
    n = jnp.tanh(gi[:, 2 * Hd:] + r * bhh[:, 2 * Hd:])
    h1 = (1.0 - z) * n
    x0 = jnp.dot(h1, We2T_ref[...], preferred_element_type=f32) + be2_ref[...]

    iotaA = jax.lax.broadcasted_iota(jnp.int32, (TN, A), 1)
    Wsh = Wsh_ref[...]
    bsh = bsh_ref[...]
    W1all = W1all_ref[...]
    WcrT = WcrT_ref[...]
    Wm2T = Wm2T_ref[...]
    bcr = bcr_ref[...]
    bm1 = bm1_ref[...]
    bm2 = bm2_ref[...]
    zeroE = jnp.zeros((TN, E), f32)

    X = [x0] + [zeroE] * (Ph - 1)
    I = jnp.zeros((TN, 1), jnp.int32)
    logits = jnp.zeros((TN, A), f32)
    sl_list = []
    ohP = None
    P = None
    for _ in range(Ps):
        x = zeroE
        for p in range(Ph):
            x = x + jnp.where(I == p, X[p], 0.0)
        rx = jnp.maximum(x, 0.0)
        sharp = (rx * Wsh).sum(axis=-1, keepdims=True) + bsh
        values = jnp.dot(rx, WcrT, preferred_element_type=f32) + bcr
        is_new = jnp.all(logits == 0.0, axis=-1, keepdims=True)
        sl = jnp.where(is_new, sharp * values, logits)
        sl_list.append(sl)
        P = jnp.argmax(sl, axis=-1).astype(jnp.int32)[:, None]
        ohP = (iotaA == P).astype(f32)
        logits = sl - INF * ohP
        v_sel = (values * ohP).sum(axis=-1, keepdims=True)
        push = v_sel > 0.0
        Y = jnp.dot(rx, W1all, preferred_element_type=f32)
        m1 = jnp.maximum(_select_action_block(Y, P) + bm1, 0.0)
        m = jnp.dot(m1, Wm2T, preferred_element_type=f32) + bm2
        I_up = jnp.minimum(I + 1, Ph - 1)
        for p in range(Ph):
            X[p] = jnp.where(push & (I_up == p), m, X[p])
        I = jnp.where(push, I_up, jnp.maximum(I - 1, 0))

    sl_out[...] = jnp.concatenate(sl_list, axis=-1)
    X_out[...] = jnp.concatenate(X, axis=-1)
    vals_cols = [(sl_list[t] * ohP).sum(axis=-1, keepdims=True) for t in range(T)]
    vals_out[...] = jnp.concatenate(vals_cols, axis=-1)

    ml_cols = []
    for t in range(T):
        rp = jnp.maximum(emb[(t - 1) % T], 0.0)
        Yt = jnp.dot(rp, W1all, preferred_element_type=f32)
        m1 = jnp.maximum(_select_action_block(Yt, P) + bm1, 0.0)
        m = jnp.dot(m1, Wm2T, preferred_element_type=f32) + bm2
        d = m - emb[t]
        ml_cols.append((d * d).mean(axis=-1, keepdims=True))
    ml_out[...] = jnp.concatenate(ml_cols, axis=-1)

    Xnorm = [jnp.sqrt((X[p] * X[p]).sum(axis=-1, keepdims=True)) for p in range(Ph)]
    el_cols = []
    for t in range(T):
        xt = emb[t]
        xtn = jnp.sqrt((xt * xt).sum(axis=-1, keepdims=True))
        sharp_t = (jnp.maximum(xt, 0.0) * Wsh).sum(axis=-1, keepdims=True) + bsh
        cos_cols = []
        for p in range(Ph):
            num = (X[p] * xt).sum(axis=-1, keepdims=True)
            cos_cols.append(num / (xtn * Xnorm[p] + 1e-8))
        s = sharp_t * jnp.concatenate(cos_cols, axis=-1)
        smax = jnp.max(s, axis=-1, keepdims=True)
        ex = jnp.exp(s - smax)
        lse = smax + jnp.log(ex.sum(axis=-1, keepdims=True))
        lp = s - lse
        el_cols.append((jnp.exp(lp) * lp).sum(axis=-1, keepdims=True))
    el_out[...] = jnp.concatenate(el_cols, axis=-1)


def kernel(obs, actions, rnn_hxs, embed1, Wih, Whh, bih, bhh, We2, be2, Wsh,
           bsh, Wcr, bcr, Wm1, bm1, Wm2, bm2):
    del actions, rnn_hxs, Whh
    obs2 = obs[:, :, 0].astype(jnp.int32).T
    W1all = Wm1.reshape(Hd, E, A).transpose(1, 2, 0).reshape(E, A * Hd)
    full = lambda a: pl.BlockSpec(a.shape, lambda i: (0,) * a.ndim)
    args = [
        embed1, Wih.T, bih[None], bhh[None], We2.T, be2[None], Wsh,
        bsh[None], Wcr.T, bcr[None], W1all, bm1[None], Wm2.T, bm2[None],
    ]
    grid = (N // TN,)
    outs = pl.pallas_call(
        _body,
        grid=grid,
        in_specs=[pl.BlockSpec((TN, T), lambda i: (i, 0))] + [full(a) for a in args],
        out_specs=[
            pl.BlockSpec((TN, Ps * A), lambda i: (i, 0)),
            pl.BlockSpec((TN, Ph * E), lambda i: (i, 0)),
            pl.BlockSpec((TN, T), lambda i: (i, 0)),
            pl.BlockSpec((TN, T), lambda i: (i, 0)),
            pl.BlockSpec((TN, T), lambda i: (i, 0)),
        ],
        out_shape=[
            jax.ShapeDtypeStruct((N, Ps * A), jnp.float32),
            jax.ShapeDtypeStruct((N, Ph * E), jnp.float32),
            jax.ShapeDtypeStruct((N, T), jnp.float32),
            jax.ShapeDtypeStruct((N, T), jnp.float32),
            jax.ShapeDtypeStruct((N, T), jnp.float32),
        ],
        compiler_params=pltpu.CompilerParams(
            dimension_semantics=("arbitrary",)),
    )(obs2, *args)
    return jnp.concatenate(outs, axis=-1)
